# baseline (speedup 1.0000x reference)
import functools

import jax
import jax.numpy as jnp
from jax import lax
from jax.experimental import pallas as pl
from jax.experimental.pallas import tpu as pltpu
from jax.experimental.pallas import tpu_sc as plsc

B, P, D, L, C = 32, 784, 2048, 256, 100
NC, NS, LN = 2, 16, 16
NW = NC * NS
DW = D // NW
WPT = DW // 4
BBLK = 8
SEG = 196
NSEG = P // SEG

_GDN = lax.GatherDimensionNumbers(
    offset_dims=(), collapsed_slice_dims=(0,), start_index_map=(0,)
)


def _vreg_take(vec, lanes):
    return lax.gather(
        vec,
        lanes[:, None],
        _GDN,
        slice_sizes=(1,),
        mode=lax.GatherScatterMode.PROMISE_IN_BOUNDS,
    )


def _pack_signs(w, rows):
    bits = (w < 0).astype(jnp.int8)
    bits = bits.reshape(rows, NW, LN, 4).transpose(1, 0, 2, 3)
    return lax.bitcast_convert_type(bits, jnp.int32)


def _sc_body(x_hbm, pos_hbm, val_hbm, out_hbm, x_v, idx_v, pos_v, val_v, enc_v, wacc_v):
    c = lax.axis_index("c")
    s = lax.axis_index("s")
    wid = s * NC + c

    pltpu.sync_copy(x_hbm, x_v)
    pltpu.sync_copy(pos_hbm.at[wid], pos_v)
    pltpu.sync_copy(val_hbm.at[wid], val_v)

    def qbody(i, _):
        v = x_v[pl.ds(i * LN, LN)] * jnp.float32(L - 1)
        t = v + jnp.float32(0.5)
        ii = t.astype(jnp.int32)
        tie = (ii.astype(jnp.float32) == t) & ((ii & 1) == 1)
        ii = jnp.where(tie, ii - 1, ii)
        ii = jnp.clip(ii, 0, L - 1)
        idx_v[pl.ds(i * LN, LN)] = ii
        return 0

    lax.fori_loop(0, B * P // LN, qbody, 0)

    iota = lax.iota(jnp.int32, LN)
    zero = jnp.zeros((LN,), jnp.int32)
    lane_sel = [jnp.full((LN,), si, jnp.int32) for si in range(BBLK)]
    byte_mask = jnp.full((LN,), 0xFF, jnp.int32)

    for bb in range(B // BBLK):
        b_flat = (jnp.int32(bb * BBLK) + (iota & jnp.int32(BBLK - 1))) * jnp.int32(P)
        for si in range(BBLK):
            for r in range(4):
                wacc_v[bb * BBLK + si, pl.ds(r * LN, LN)] = zero

        for seg in range(NSEG):
            def pbody(p, accs):
                idxrow = plsc.load_gather(idx_v, [b_flat + p])
                posw = pos_v[p, :]
                out = []
                for si in range(BBLK):
                    rbase = _vreg_take(idxrow, lane_sel[si])
                    valw = plsc.load_gather(val_v, [rbase, iota])
                    out.append(accs[si] + (valw ^ posw))
                return tuple(out)

            accs = lax.fori_loop(
                seg * SEG,
                (seg + 1) * SEG,
                pbody,
                tuple(zero for _ in range(BBLK)),
            )
            for si in range(BBLK):
                acc = accs[si]
                brow = bb * BBLK + si
                for r in range(4):
                    cnt = (lax.shift_right_logical(acc, jnp.int32(8 * r))
                           & byte_mask)
                    wacc_v[brow, pl.ds(r * LN, LN)] = (
                        wacc_v[brow, pl.ds(r * LN, LN)] + cnt
                    )

    half = jnp.int32(P // 2)
    col_sel = [iota * 4 + jnp.int32(r) for r in range(4)]
    for brow in range(B):
        for r in range(4):
            cnt = wacc_v[brow, pl.ds(r * LN, LN)]
            e = jnp.where(cnt < half, jnp.float32(1), jnp.float32(-1))
            plsc.store_scatter(
                enc_v, [jnp.full((LN,), brow, jnp.int32), col_sel[r]], e
            )

    pltpu.sync_copy(enc_v, out_hbm.at[:, pl.ds(wid * DW, DW)])


@jax.jit
def _sc_encode(xf, pos_pk, val_pk):
    mesh = plsc.VectorSubcoreMesh(core_axis_name="c", subcore_axis_name="s")
    f = functools.partial(
        pl.kernel,
        out_type=jax.ShapeDtypeStruct((B, D), jnp.float32),
        mesh=mesh,
        compiler_params=pltpu.CompilerParams(
            use_tc_tiling_on_sc=False, needs_layout_passes=False
        ),
        scratch_types=[
            pltpu.VMEM((B * P,), jnp.float32),
            pltpu.VMEM((B * P,), jnp.int32),
            pltpu.VMEM((P, WPT), jnp.int32),
            pltpu.VMEM((L, WPT), jnp.int32),
            pltpu.VMEM((B, DW), jnp.float32),
            pltpu.VMEM((B, DW), jnp.int32),
        ],
    )(_sc_body)
    return f(xf, pos_pk, val_pk)


def _classify_body(enc_ref, w_ref, out_ref):
    out_ref[...] = lax.dot_general(
        enc_ref[...],
        w_ref[...],
        (((1,), (1,)), ((), ())),
        preferred_element_type=jnp.float32,
    )


@jax.jit
def _classify(enc, classify_weight):
    return pl.pallas_call(
        _classify_body,
        out_shape=jax.ShapeDtypeStruct((B, C), jnp.float32),
    )(enc, classify_weight)


def kernel(x, position_weight, value_weight, classify_weight):
    xf = x.reshape(B * P)
    pos_pk = _pack_signs(position_weight, P)
    val_pk = _pack_signs(value_weight, L)
    enc = _sc_encode(xf, pos_pk, val_pk)
    return _classify(enc, classify_weight)


# speedup vs baseline: 1.3497x; 1.3497x over previous
import functools

import jax
import jax.numpy as jnp
from jax import lax
from jax.experimental import pallas as pl
from jax.experimental.pallas import tpu as pltpu
from jax.experimental.pallas import tpu_sc as plsc

B, P, D, L, C = 32, 784, 2048, 256, 100
NC, NS, LN = 2, 16, 16
NW = NC * NS
DW = D // NW
WPT = DW // 4
BBLK = 8
SEG = 196
NSEG = P // SEG

_GDN = lax.GatherDimensionNumbers(
    offset_dims=(), collapsed_slice_dims=(0,), start_index_map=(0,)
)


def _vreg_take(vec, lanes):
    return lax.gather(
        vec,
        lanes[:, None],
        _GDN,
        slice_sizes=(1,),
        mode=lax.GatherScatterMode.PROMISE_IN_BOUNDS,
    )


def _pack_signs(w, rows):
    bits = (w < 0).astype(jnp.int8)
    bits = bits.reshape(rows, NW, 4, LN).transpose(0, 1, 3, 2)
    return lax.bitcast_convert_type(bits, jnp.int32).reshape(rows, D // 4)


def _sc_body(x_hbm, pos_hbm, val_hbm, out_hbm, x_v, idx_v, pos_v, val_v, enc_v, wacc_v):
    c = lax.axis_index("c")
    s = lax.axis_index("s")
    wid = s * NC + c
    c0 = wid * WPT

    pltpu.sync_copy(x_hbm, x_v)
    pltpu.sync_copy(pos_hbm.at[:, pl.ds(c0, WPT)], pos_v)
    pltpu.sync_copy(val_hbm.at[:, pl.ds(c0, WPT)], val_v)

    def qbody(i, _):
        v = x_v[pl.ds(i * LN, LN)] * jnp.float32(L - 1)
        t = v + jnp.float32(0.5)
        ii = t.astype(jnp.int32)
        tie = (ii.astype(jnp.float32) == t) & ((ii & 1) == 1)
        ii = jnp.where(tie, ii - 1, ii)
        ii = jnp.clip(ii, 0, L - 1)
        idx_v[pl.ds(i * LN, LN)] = ii
        return 0

    lax.fori_loop(0, B * P // LN, qbody, 0)

    iota = lax.iota(jnp.int32, LN)
    zero = jnp.zeros((LN,), jnp.int32)
    lane_sel = [jnp.full((LN,), si, jnp.int32) for si in range(BBLK)]
    byte_mask = jnp.full((LN,), 0xFF, jnp.int32)

    for bb in range(B // BBLK):
        b_flat = (jnp.int32(bb * BBLK) + (iota & jnp.int32(BBLK - 1))) * jnp.int32(P)
        for si in range(BBLK):
            for r in range(4):
                wacc_v[bb * BBLK + si, pl.ds(r * LN, LN)] = zero

        for seg in range(NSEG):
            def pbody(p, accs):
                idxrow = plsc.load_gather(idx_v, [b_flat + p])
                posw = pos_v[p, :]
                out = []
                for si in range(BBLK):
                    rbase = _vreg_take(idxrow, lane_sel[si])
                    valw = plsc.load_gather(val_v, [rbase, iota])
                    out.append(accs[si] + (valw ^ posw))
                return tuple(out)

            accs = lax.fori_loop(
                seg * SEG,
                (seg + 1) * SEG,
                pbody,
                tuple(zero for _ in range(BBLK)),
            )
            for si in range(BBLK):
                acc = accs[si]
                brow = bb * BBLK + si
                for r in range(4):
                    cnt = (lax.shift_right_logical(acc, jnp.int32(8 * r))
                           & byte_mask)
                    wacc_v[brow, pl.ds(r * LN, LN)] = (
                        wacc_v[brow, pl.ds(r * LN, LN)] + cnt
                    )

    half = jnp.int32(P // 2)
    for brow in range(B):
        for r in range(4):
            cnt = wacc_v[brow, pl.ds(r * LN, LN)]
            enc_v[brow, pl.ds(r * LN, LN)] = jnp.where(
                cnt < half, jnp.float32(1), jnp.float32(-1)
            )

    pltpu.sync_copy(enc_v, out_hbm.at[:, pl.ds(wid * DW, DW)])


@jax.jit
def _sc_encode(xf, pos_pk, val_pk):
    mesh = plsc.VectorSubcoreMesh(core_axis_name="c", subcore_axis_name="s")
    f = functools.partial(
        pl.kernel,
        out_type=jax.ShapeDtypeStruct((B, D), jnp.float32),
        mesh=mesh,
        compiler_params=pltpu.CompilerParams(
            use_tc_tiling_on_sc=False, needs_layout_passes=False
        ),
        scratch_types=[
            pltpu.VMEM((B * P,), jnp.float32),
            pltpu.VMEM((B * P,), jnp.int32),
            pltpu.VMEM((P, WPT), jnp.int32),
            pltpu.VMEM((L, WPT), jnp.int32),
            pltpu.VMEM((B, DW), jnp.float32),
            pltpu.VMEM((B, DW), jnp.int32),
        ],
    )(_sc_body)
    return f(xf, pos_pk, val_pk)


def _classify_body(enc_ref, w_ref, out_ref):
    out_ref[...] = lax.dot_general(
        enc_ref[...],
        w_ref[...],
        (((1,), (1,)), ((), ())),
        preferred_element_type=jnp.float32,
    )


@jax.jit
def _classify(enc, classify_weight):
    return pl.pallas_call(
        _classify_body,
        out_shape=jax.ShapeDtypeStruct((B, C), jnp.float32),
    )(enc, classify_weight)


def kernel(x, position_weight, value_weight, classify_weight):
    xf = x.reshape(B * P)
    pos_pk = _pack_signs(position_weight, P)
    val_pk = _pack_signs(value_weight, L)
    enc = _sc_encode(xf, pos_pk, val_pk)
    return _classify(enc, classify_weight)


# speedup vs baseline: 1.4874x; 1.1020x over previous
import functools

import jax
import jax.numpy as jnp
from jax import lax
from jax.experimental import pallas as pl
from jax.experimental.pallas import tpu as pltpu
from jax.experimental.pallas import tpu_sc as plsc

B, P, D, L, C = 32, 784, 2048, 256, 100
NC, NS, LN = 2, 16, 16
NW = NC * NS
DW = D // NW
WPT = DW // 4
BBLK = 8
SEG = 196
NSEG = P // SEG

_GDN = lax.GatherDimensionNumbers(
    offset_dims=(), collapsed_slice_dims=(0,), start_index_map=(0,)
)


def _vreg_take(vec, lanes):
    return lax.gather(
        vec,
        lanes[:, None],
        _GDN,
        slice_sizes=(1,),
        mode=lax.GatherScatterMode.PROMISE_IN_BOUNDS,
    )


def _sc_body(x_hbm, pos_hbm, val_hbm, out_hbm, x_v, idx_v, posraw_v, valraw_v,
             val_v, enc_v, wacc_v, sem):
    c = lax.axis_index("c")
    s = lax.axis_index("s")
    wid = s * NC + c
    cb = wid // 2
    h = (wid % 2) * DW

    cp_pos = pltpu.async_copy(
        pos_hbm.at[:, cb, :, pl.ds(h, DW)], posraw_v, sem
    )
    pltpu.sync_copy(x_hbm, x_v)

    def qbody(i, _):
        v = x_v[pl.ds(i * LN, LN)] * jnp.float32(L - 1)
        t = v + jnp.float32(0.5)
        ii = t.astype(jnp.int32)
        tie = (ii.astype(jnp.float32) == t) & ((ii & 1) == 1)
        ii = jnp.where(tie, ii - 1, ii)
        ii = jnp.clip(ii, 0, L - 1)
        idx_v[pl.ds(i * LN, LN)] = ii
        return 0

    lax.fori_loop(0, B * P // LN, qbody, 0)
    cp_pos.wait()
    cp_val = pltpu.async_copy(
        val_hbm.at[:, cb, :, pl.ds(h, DW)], valraw_v, sem
    )

    one = jnp.full((LN,), 1, jnp.int32)
    zero16 = jnp.zeros((LN,), jnp.int32)

    def _pack_row(raw_ref, rb, r):
        w = zero16
        for q in range(4):
            v = raw_ref[rb, r, pl.ds(q * LN, LN)]
            bit = jnp.where(v < 0, one, zero16)
            w = w | (bit << (8 * q))
        return w

    def posbody(p, _):
        w = _pack_row(posraw_v, p >> 3, p & 7)
        x_v[pl.ds(p * WPT, WPT)] = plsc.bitcast(w, jnp.float32)
        return 0

    lax.fori_loop(0, P, posbody, 0)
    cp_val.wait()

    def valbody(l, _):
        val_v[l, :] = _pack_row(valraw_v, l >> 3, l & 7)
        return 0

    lax.fori_loop(0, L, valbody, 0)

    iota = lax.iota(jnp.int32, LN)
    zero = jnp.zeros((LN,), jnp.int32)
    lane_sel = [jnp.full((LN,), si, jnp.int32) for si in range(BBLK)]
    byte_mask = jnp.full((LN,), 0xFF, jnp.int32)

    for bb in range(B // BBLK):
        b_flat = (jnp.int32(bb * BBLK) + (iota & jnp.int32(BBLK - 1))) * jnp.int32(P)
        for si in range(BBLK):
            for r in range(4):
                wacc_v[bb * BBLK + si, pl.ds(r * LN, LN)] = zero

        for seg in range(NSEG):
            def pbody(p, accs):
                idxrow = plsc.load_gather(idx_v, [b_flat + p])
                posw = plsc.bitcast(x_v[pl.ds(p * WPT, WPT)], jnp.int32)
                out = []
                for si in range(BBLK):
                    rbase = _vreg_take(idxrow, lane_sel[si])
                    valw = plsc.load_gather(val_v, [rbase, iota])
                    out.append(accs[si] + (valw ^ posw))
                return tuple(out)

            accs = lax.fori_loop(
                seg * SEG,
                (seg + 1) * SEG,
                pbody,
                tuple(zero for _ in range(BBLK)),
            )
            for si in range(BBLK):
                acc = accs[si]
                brow = bb * BBLK + si
                for r in range(4):
                    cnt = (lax.shift_right_logical(acc, jnp.int32(8 * r))
                           & byte_mask)
                    wacc_v[brow, pl.ds(r * LN, LN)] = (
                        wacc_v[brow, pl.ds(r * LN, LN)] + cnt
                    )

    half = jnp.int32(P // 2)
    for brow in range(B):
        for r in range(4):
            cnt = wacc_v[brow, pl.ds(r * LN, LN)]
            enc_v[brow, pl.ds(r * LN, LN)] = jnp.where(
                cnt < half, jnp.float32(1), jnp.float32(-1)
            )

    pltpu.sync_copy(enc_v, out_hbm.at[:, pl.ds(wid * DW, DW)])


@jax.jit
def _sc_encode(xf, pos4, val4):
    mesh = plsc.VectorSubcoreMesh(core_axis_name="c", subcore_axis_name="s")
    f = functools.partial(
        pl.kernel,
        out_type=jax.ShapeDtypeStruct((B, D), jnp.float32),
        mesh=mesh,
        compiler_params=pltpu.CompilerParams(
            use_tc_tiling_on_sc=False, needs_layout_passes=False
        ),
        scratch_types=[
            pltpu.VMEM((B * P,), jnp.float32),
            pltpu.VMEM((B * P,), jnp.int32),
            pltpu.VMEM((P // 8, 8, DW), jnp.float32),
            pltpu.VMEM((L // 8, 8, DW), jnp.float32),
            pltpu.VMEM((L, WPT), jnp.int32),
            pltpu.VMEM((B, DW), jnp.float32),
            pltpu.VMEM((B, DW), jnp.int32),
            pltpu.SemaphoreType.DMA,
        ],
    )(_sc_body)
    return f(xf, pos4, val4)


def _classify_body(enc_ref, w_ref, out_ref):
    out_ref[...] = lax.dot_general(
        enc_ref[...],
        w_ref[...],
        (((1,), (1,)), ((), ())),
        preferred_element_type=jnp.float32,
    )


@jax.jit
def _classify(enc, classify_weight):
    return pl.pallas_call(
        _classify_body,
        out_shape=jax.ShapeDtypeStruct((B, C), jnp.float32),
    )(enc, classify_weight)


def kernel(x, position_weight, value_weight, classify_weight):
    xf = x.reshape(B * P)
    pos4 = position_weight.reshape(P // 8, 8, LN, 128).transpose(0, 2, 1, 3)
    val4 = value_weight.reshape(L // 8, 8, LN, 128).transpose(0, 2, 1, 3)
    enc = _sc_encode(xf, pos4, val4)
    return _classify(enc, classify_weight)


# speedup vs baseline: 1.5178x; 1.0205x over previous
import functools

import jax
import jax.numpy as jnp
from jax import lax
from jax.experimental import pallas as pl
from jax.experimental.pallas import tpu as pltpu
from jax.experimental.pallas import tpu_sc as plsc

B, P, D, L, C = 32, 784, 2048, 256, 100
NC, NS, LN = 2, 16, 16
NW = NC * NS
DW = D // NW
WPT = DW // 4
BBLK = 8
SEG = 196
NSEG = P // SEG

_GDN = lax.GatherDimensionNumbers(
    offset_dims=(), collapsed_slice_dims=(0,), start_index_map=(0,)
)


def _vreg_take(vec, lanes):
    return lax.gather(
        vec,
        lanes[:, None],
        _GDN,
        slice_sizes=(1,),
        mode=lax.GatherScatterMode.PROMISE_IN_BOUNDS,
    )


def _sc_body(x_hbm, pos_hbm, val_hbm, out_hbm, x_v, idx_v, posraw_v, valraw_v,
             val_v, enc_v, wacc_v, sem):
    c = lax.axis_index("c")
    s = lax.axis_index("s")
    wid = s * NC + c
    cb = wid // 2
    h = (wid % 2) * DW

    cp_pos = pltpu.async_copy(
        pos_hbm.at[:, cb, :, pl.ds(h, DW)], posraw_v, sem
    )
    pltpu.sync_copy(x_hbm, x_v)

    def qbody(i, _):
        v = x_v[pl.ds(i * LN, LN)] * jnp.float32(L - 1)
        t = v + jnp.float32(0.5)
        ii = t.astype(jnp.int32)
        tie = (ii.astype(jnp.float32) == t) & ((ii & 1) == 1)
        ii = jnp.where(tie, ii - 1, ii)
        ii = jnp.clip(ii, 0, L - 1)
        idx_v[pl.ds(i * LN, LN)] = ii
        return 0

    lax.fori_loop(0, B * P // LN, qbody, 0)
    cp_pos.wait()
    cp_val = pltpu.async_copy(
        val_hbm.at[:, cb, :, pl.ds(h, DW)], valraw_v, sem
    )

    one = jnp.full((LN,), 1, jnp.int32)
    zero16 = jnp.zeros((LN,), jnp.int32)

    def _pack_row(raw_ref, rb, r):
        w = zero16
        for q in range(4):
            v = raw_ref[rb, r, pl.ds(q * LN, LN)]
            bit = jnp.where(v < 0, one, zero16)
            w = w | (bit << (8 * q))
        return w

    def posbody(p, _):
        w = _pack_row(posraw_v, p >> 3, p & 7)
        x_v[pl.ds(p * WPT, WPT)] = plsc.bitcast(w, jnp.float32)
        return 0

    lax.fori_loop(0, P, posbody, 0)
    cp_val.wait()

    def valbody(l, _):
        val_v[l, :] = _pack_row(valraw_v, l >> 3, l & 7)
        return 0

    lax.fori_loop(0, L, valbody, 0)

    iota = lax.iota(jnp.int32, LN)
    zero = jnp.zeros((LN,), jnp.int32)
    lane_sel = [jnp.full((LN,), si, jnp.int32) for si in range(BBLK)]
    byte_mask = jnp.full((LN,), 0xFF, jnp.int32)

    for bb in range(B // BBLK):
        b_flat = (jnp.int32(bb * BBLK) + (iota & jnp.int32(BBLK - 1))) * jnp.int32(P)
        for si in range(BBLK):
            for r in range(4):
                wacc_v[bb * BBLK + si, pl.ds(r * LN, LN)] = zero

        for seg in range(NSEG):
            def pbody(p, accs):
                idxrow = plsc.load_gather(idx_v, [b_flat + p])
                posw = plsc.bitcast(x_v[pl.ds(p * WPT, WPT)], jnp.int32)
                out = []
                for si in range(BBLK):
                    rbase = _vreg_take(idxrow, lane_sel[si])
                    valw = plsc.load_gather(val_v, [rbase, iota])
                    out.append(accs[si] + (valw ^ posw))
                return tuple(out)

            accs = lax.fori_loop(
                seg * SEG,
                (seg + 1) * SEG,
                pbody,
                tuple(zero for _ in range(BBLK)),
            )
            for si in range(BBLK):
                acc = accs[si]
                brow = bb * BBLK + si
                for r in range(4):
                    cnt = (lax.shift_right_logical(acc, jnp.int32(8 * r))
                           & byte_mask)
                    wacc_v[brow, pl.ds(r * LN, LN)] = (
                        wacc_v[brow, pl.ds(r * LN, LN)] + cnt
                    )

    half = jnp.int32(P // 2)
    for brow in range(B):
        for r in range(4):
            cnt = wacc_v[brow, pl.ds(r * LN, LN)]
            enc_v[brow >> 3, brow & 7, pl.ds(r * LN, LN)] = jnp.where(
                cnt < half, jnp.float32(1), jnp.float32(-1)
            )

    pltpu.sync_copy(enc_v, out_hbm.at[:, cb, :, pl.ds(h, DW)])


@jax.jit
def _sc_encode(xf, pos4, val4):
    mesh = plsc.VectorSubcoreMesh(core_axis_name="c", subcore_axis_name="s")
    f = functools.partial(
        pl.kernel,
        out_type=jax.ShapeDtypeStruct((B // 8, LN, 8, 128), jnp.float32),
        mesh=mesh,
        compiler_params=pltpu.CompilerParams(
            use_tc_tiling_on_sc=False, needs_layout_passes=False
        ),
        scratch_types=[
            pltpu.VMEM((B * P,), jnp.float32),
            pltpu.VMEM((B * P,), jnp.int32),
            pltpu.VMEM((P // 8, 8, DW), jnp.float32),
            pltpu.VMEM((L // 8, 8, DW), jnp.float32),
            pltpu.VMEM((L, WPT), jnp.int32),
            pltpu.VMEM((B // 8, 8, DW), jnp.float32),
            pltpu.VMEM((B, DW), jnp.int32),
            pltpu.SemaphoreType.DMA,
        ],
    )(_sc_body)
    return f(xf, pos4, val4)


def _classify_body(enc_ref, w_ref, out_ref):
    out_ref[...] = lax.dot_general(
        enc_ref[...],
        w_ref[...],
        (((1,), (1,)), ((), ())),
        preferred_element_type=jnp.float32,
    )


@jax.jit
def _classify(enc, classify_weight):
    return pl.pallas_call(
        _classify_body,
        out_shape=jax.ShapeDtypeStruct((B, C), jnp.float32),
    )(enc, classify_weight)


def kernel(x, position_weight, value_weight, classify_weight):
    xf = x.reshape(B * P)
    pos4 = position_weight.reshape(P // 8, 8, LN, 128).transpose(0, 2, 1, 3)
    val4 = value_weight.reshape(L // 8, 8, LN, 128).transpose(0, 2, 1, 3)
    enc4 = _sc_encode(xf, pos4, val4)
    enc = enc4.transpose(0, 2, 1, 3).reshape(B, D)
    return _classify(enc, classify_weight)


# speedup vs baseline: 1.5446x; 1.0176x over previous
import functools

import numpy as np

import jax
import jax.numpy as jnp
from jax import lax
from jax.experimental import pallas as pl
from jax.experimental.pallas import tpu as pltpu
from jax.experimental.pallas import tpu_sc as plsc

B, P, D, L, C = 32, 784, 2048, 256, 100
NC, NS, LN = 2, 16, 16
NW = NC * NS
DW = D // NW
BBLK = 8
NPAIR = BBLK // 2
L1N, L2N, L2C = 14, 14, 4

_GDN = lax.GatherDimensionNumbers(
    offset_dims=(), collapsed_slice_dims=(0,), start_index_map=(0,)
)


def _vreg_take(vec, lanes):
    return lax.gather(
        vec,
        lanes[:, None],
        _GDN,
        slice_sizes=(1,),
        mode=lax.GatherScatterMode.PROMISE_IN_BOUNDS,
    )


def _sc_body(x_hbm, pos_hbm, val_hbm, out_hbm, x_v, idx_v, posraw_v, valraw_v,
             val_v, enc_v, wacc_v, sem):
    c = lax.axis_index("c")
    s = lax.axis_index("s")
    wid = s * NC + c
    cb = wid // 2
    h = (wid % 2) * DW

    PH = P // 16
    cp_pos = pltpu.async_copy(
        pos_hbm.at[pl.ds(0, PH), cb, :, pl.ds(h, DW)], posraw_v, sem
    )
    pltpu.sync_copy(x_hbm, x_v)

    def qbody(i, _):
        v = x_v[pl.ds(i * LN, LN)] * jnp.float32(L - 1)
        t = v + jnp.float32(0.5)
        ii = t.astype(jnp.int32)
        tie = (ii.astype(jnp.float32) == t) & ((ii & 1) == 1)
        ii = jnp.where(tie, ii - 1, ii)
        ii = jnp.clip(ii, 0, L - 1)
        idx_v[pl.ds(i * LN, LN)] = ii
        return 0

    lax.fori_loop(0, B * P // LN, qbody, 0)
    cp_pos.wait()
    cp_val = pltpu.async_copy(
        val_hbm.at[:, cb, :, pl.ds(h, DW)], valraw_v, sem
    )
    PHALF = P // 2

    iota = lax.iota(jnp.int32, LN)
    one = jnp.full((LN,), 1, jnp.int32)
    zero16 = jnp.zeros((LN,), jnp.int32)
    swap_pat = (iota + 8) & 15
    dup_pat = iota & 7

    def _pack_row(raw_ref, rb, r):
        w = zero16
        for n2 in range(4):
            v = raw_ref[rb, r, pl.ds(n2 * LN, LN)]
            bit = jnp.where(v < 0, one, zero16)
            nib = bit | (_vreg_take(bit, swap_pat) << 4)
            w = w | (nib << (8 * n2))
        return _vreg_take(w, dup_pat)

    def posbody(p, _):
        w = _pack_row(posraw_v, p >> 3, p & 7)
        x_v[pl.ds(p * LN, LN)] = plsc.bitcast(w, jnp.float32)
        return 0

    lax.fori_loop(0, PHALF, posbody, 0)
    cp_pos2 = pltpu.async_copy(
        pos_hbm.at[pl.ds(PH, PH), cb, :, pl.ds(h, DW)], posraw_v, sem
    )
    cp_pos2.wait()

    def posbody2(p, _):
        w = _pack_row(posraw_v, (p >> 3) - PH, p & 7)
        x_v[pl.ds(p * LN, LN)] = plsc.bitcast(w, jnp.float32)
        return 0

    lax.fori_loop(PHALF, P, posbody2, 0)
    cp_val.wait()

    def valbody(l, _):
        val_v[l, :] = _pack_row(valraw_v, l >> 3, l & 7)
        return 0

    lax.fori_loop(0, L, valbody, 0)

    nib_mask = jnp.full((LN,), 0x0F0F0F0F, jnp.int32)
    byte_mask = jnp.full((LN,), 0xFF, jnp.int32)
    halfsel = iota >> 3
    pair_pat = [jnp.int32(2 * j) + halfsel for j in range(NPAIR)]

    for bb in range(B // BBLK):
        b_flat = (jnp.int32(bb * BBLK) + (iota & jnp.int32(BBLK - 1))) * jnp.int32(P)
        for j in range(NPAIR):
            for n in range(8):
                wacc_v[bb * NPAIR + j, n, :] = zero16

        def l2body(l2, _):
            def l1body(l1, byteaccs):
                base = l2 * (L1N * L2N) + l1 * L1N

                def pbody(i, nibaccs):
                    p = base + i
                    idxrow = plsc.load_gather(idx_v, [b_flat + p])
                    posw = plsc.bitcast(x_v[pl.ds(p * LN, LN)], jnp.int32)
                    out = []
                    for j in range(NPAIR):
                        rbp = _vreg_take(idxrow, pair_pat[j])
                        valw = plsc.load_gather(val_v, [rbp, iota])
                        out.append(nibaccs[j] + (valw ^ posw))
                    return tuple(out)

                nib = lax.fori_loop(
                    0, L1N, pbody, tuple(zero16 for _ in range(NPAIR))
                )
                out = []
                for j in range(NPAIR):
                    lo, hi = byteaccs[2 * j], byteaccs[2 * j + 1]
                    out.append(lo + (nib[j] & nib_mask))
                    out.append(hi + ((nib[j] >> 4) & nib_mask))
                return tuple(out)

            byteaccs = lax.fori_loop(
                0, L2N, l1body, tuple(zero16 for _ in range(2 * NPAIR))
            )
            for j in range(NPAIR):
                row = bb * NPAIR + j
                lo, hi = byteaccs[2 * j], byteaccs[2 * j + 1]
                for r in range(4):
                    wacc_v[row, 2 * r, :] = (
                        wacc_v[row, 2 * r, :]
                        + (lax.shift_right_logical(lo, jnp.int32(8 * r)) & byte_mask)
                    )
                    wacc_v[row, 2 * r + 1, :] = (
                        wacc_v[row, 2 * r + 1, :]
                        + (lax.shift_right_logical(hi, jnp.int32(8 * r)) & byte_mask)
                    )
            return 0

        lax.fori_loop(0, L2C, l2body, 0)

    half = jnp.int32(P // 2)
    col8 = iota & 7
    for q in range(B // 2):
        bvec = jnp.int32(2 * q) + halfsel
        rb = bvec >> 3
        rr = bvec & 7
        for n in range(8):
            cnt = wacc_v[q, n, :]
            e = jnp.where(cnt < half, jnp.float32(1), jnp.float32(-1))
            plsc.store_scatter(enc_v, [rb, rr, col8 + jnp.int32(8 * n)], e)

    pltpu.sync_copy(enc_v, out_hbm.at[:, cb, :, pl.ds(h, DW)])


@jax.jit
def _sc_encode(xf, pos4, val4):
    mesh = plsc.VectorSubcoreMesh(core_axis_name="c", subcore_axis_name="s")
    f = functools.partial(
        pl.kernel,
        out_type=jax.ShapeDtypeStruct((B // 8, LN, 8, 128), jnp.float32),
        mesh=mesh,
        compiler_params=pltpu.CompilerParams(
            use_tc_tiling_on_sc=False, needs_layout_passes=False
        ),
        scratch_types=[
            pltpu.VMEM((B * P,), jnp.float32),
            pltpu.VMEM((B * P,), jnp.int32),
            pltpu.VMEM((P // 16, 8, DW), jnp.float32),
            pltpu.VMEM((L // 8, 8, DW), jnp.float32),
            pltpu.VMEM((L, LN), jnp.int32),
            pltpu.VMEM((B // 8, 8, DW), jnp.float32),
            pltpu.VMEM((B // 2, 8, LN), jnp.int32),
            pltpu.SemaphoreType.DMA,
        ],
    )(_sc_body)
    return f(xf, pos4, val4)


def _classify_body(enc_ref, w_ref, out_ref):
    out_ref[...] = lax.dot_general(
        enc_ref[...],
        w_ref[...],
        (((1,), (1,)), ((), ())),
        preferred_element_type=jnp.float32,
    )


@jax.jit
def _classify(enc, classify_weight):
    return pl.pallas_call(
        _classify_body,
        out_shape=jax.ShapeDtypeStruct((B, C), jnp.float32),
    )(enc, classify_weight)


def kernel(x, position_weight, value_weight, classify_weight):
    xf = x.reshape(B * P)
    pos4 = position_weight.reshape(P // 8, 8, LN, 128).transpose(0, 2, 1, 3)
    val4 = value_weight.reshape(L // 8, 8, LN, 128).transpose(0, 2, 1, 3)
    enc4 = _sc_encode(xf, pos4, val4)
    enc = enc4.transpose(0, 2, 1, 3).reshape(B, D)
    return _classify(enc, classify_weight)


# speedup vs baseline: 1.7150x; 1.1103x over previous
import functools

import numpy as np

import jax
import jax.numpy as jnp
from jax import lax
from jax.experimental import pallas as pl
from jax.experimental.pallas import tpu as pltpu
from jax.experimental.pallas import tpu_sc as plsc

B, P, D, L, C = 32, 784, 2048, 256, 100
NC, NS, LN = 2, 16, 16
NW = NC * NS
DW = D // NW
BBLK = 8
NPAIR = BBLK // 2
L1N, L2N, L2C = 14, 14, 4

_GDN = lax.GatherDimensionNumbers(
    offset_dims=(), collapsed_slice_dims=(0,), start_index_map=(0,)
)


def _vreg_take(vec, lanes):
    return lax.gather(
        vec,
        lanes[:, None],
        _GDN,
        slice_sizes=(1,),
        mode=lax.GatherScatterMode.PROMISE_IN_BOUNDS,
    )


def _sc_body(x_hbm, pos_hbm, val_hbm, out_hbm, xs_v, qtmp_v, posw_v, idx_v,
             posraw_v, valraw_v, val_v, enc_v, wacc_v, spidx_v, sem):
    c = lax.axis_index("c")
    s = lax.axis_index("s")
    wid = s * NC + c
    cb = wid // 2
    h = (wid % 2) * DW

    PH = P // 16
    cp_pos = pltpu.async_copy(
        pos_hbm.at[pl.ds(0, PH), cb, :, pl.ds(h, DW)], posraw_v, sem
    )

    XS = B * P // NS
    pltpu.sync_copy(x_hbm.at[pl.ds(s * XS, XS)], xs_v)

    def qbody(i, _):
        v = xs_v[pl.ds(i * LN, LN)] * jnp.float32(L - 1)
        t = v + jnp.float32(0.5)
        ii = t.astype(jnp.int32)
        tie = (ii.astype(jnp.float32) == t) & ((ii & 1) == 1)
        ii = jnp.where(tie, ii - 1, ii)
        ii = jnp.clip(ii, 0, L - 1)
        qtmp_v[pl.ds(i * LN, LN)] = ii
        return 0

    lax.fori_loop(0, XS // LN, qbody, 0)
    pltpu.sync_copy(qtmp_v, spidx_v.at[pl.ds(s * XS, XS)])
    plsc.subcore_barrier()
    pltpu.sync_copy(spidx_v, idx_v)
    cp_pos.wait()
    cp_val = pltpu.async_copy(
        val_hbm.at[:, cb, :, pl.ds(h, DW)], valraw_v, sem
    )
    PHALF = P // 2

    iota = lax.iota(jnp.int32, LN)
    one = jnp.full((LN,), 1, jnp.int32)
    zero16 = jnp.zeros((LN,), jnp.int32)
    swap_pat = (iota + 8) & 15
    dup_pat = iota & 7

    def _pack_row(raw_ref, rb, r):
        w = zero16
        for n2 in range(4):
            v = raw_ref[rb, r, pl.ds(n2 * LN, LN)]
            bit = jnp.where(v < 0, one, zero16)
            nib = bit | (_vreg_take(bit, swap_pat) << 4)
            w = w | (nib << (8 * n2))
        return _vreg_take(w, dup_pat)

    def posbody(p, _):
        posw_v[pl.ds(p * LN, LN)] = _pack_row(posraw_v, p >> 3, p & 7)
        return 0

    lax.fori_loop(0, PHALF, posbody, 0)
    cp_pos2 = pltpu.async_copy(
        pos_hbm.at[pl.ds(PH, PH), cb, :, pl.ds(h, DW)], posraw_v, sem
    )
    cp_pos2.wait()

    def posbody2(p, _):
        posw_v[pl.ds(p * LN, LN)] = _pack_row(posraw_v, (p >> 3) - PH, p & 7)
        return 0

    lax.fori_loop(PHALF, P, posbody2, 0)
    cp_val.wait()

    def valbody(l, _):
        val_v[l, :] = _pack_row(valraw_v, l >> 3, l & 7)
        return 0

    lax.fori_loop(0, L, valbody, 0)

    nib_mask = jnp.full((LN,), 0x0F0F0F0F, jnp.int32)
    byte_mask = jnp.full((LN,), 0xFF, jnp.int32)
    halfsel = iota >> 3
    pair_pat = [jnp.int32(2 * j) + halfsel for j in range(NPAIR)]

    for bb in range(B // BBLK):
        b_flat = (jnp.int32(bb * BBLK) + (iota & jnp.int32(BBLK - 1))) * jnp.int32(P)
        for j in range(NPAIR):
            for n in range(8):
                wacc_v[bb * NPAIR + j, n, :] = zero16

        def l2body(l2, _):
            def l1body(l1, byteaccs):
                base = l2 * (L1N * L2N) + l1 * L1N

                def pbody(i, nibaccs):
                    p = base + i
                    idxrow = plsc.load_gather(idx_v, [b_flat + p])
                    posw = posw_v[pl.ds(p * LN, LN)]
                    out = []
                    for j in range(NPAIR):
                        rbp = _vreg_take(idxrow, pair_pat[j])
                        valw = plsc.load_gather(val_v, [rbp, iota])
                        out.append(nibaccs[j] + (valw ^ posw))
                    return tuple(out)

                nib = lax.fori_loop(
                    0, L1N, pbody, tuple(zero16 for _ in range(NPAIR))
                )
                out = []
                for j in range(NPAIR):
                    lo, hi = byteaccs[2 * j], byteaccs[2 * j + 1]
                    out.append(lo + (nib[j] & nib_mask))
                    out.append(hi + ((nib[j] >> 4) & nib_mask))
                return tuple(out)

            byteaccs = lax.fori_loop(
                0, L2N, l1body, tuple(zero16 for _ in range(2 * NPAIR))
            )
            for j in range(NPAIR):
                row = bb * NPAIR + j
                lo, hi = byteaccs[2 * j], byteaccs[2 * j + 1]
                for r in range(4):
                    wacc_v[row, 2 * r, :] = (
                        wacc_v[row, 2 * r, :]
                        + (lax.shift_right_logical(lo, jnp.int32(8 * r)) & byte_mask)
                    )
                    wacc_v[row, 2 * r + 1, :] = (
                        wacc_v[row, 2 * r + 1, :]
                        + (lax.shift_right_logical(hi, jnp.int32(8 * r)) & byte_mask)
                    )
            return 0

        lax.fori_loop(0, L2C, l2body, 0)

    half = jnp.int32(P // 2)
    col8 = iota & 7
    for q in range(B // 2):
        bvec = jnp.int32(2 * q) + halfsel
        rb = bvec >> 3
        rr = bvec & 7
        for n in range(8):
            cnt = wacc_v[q, n, :]
            e = jnp.where(cnt < half, jnp.float32(1), jnp.float32(-1))
            plsc.store_scatter(enc_v, [rb, rr, col8 + jnp.int32(8 * n)], e)

    pltpu.sync_copy(enc_v, out_hbm.at[:, cb, :, pl.ds(h, DW)])


@jax.jit
def _sc_encode(xf, pos4, val4):
    mesh = plsc.VectorSubcoreMesh(core_axis_name="c", subcore_axis_name="s")
    f = functools.partial(
        pl.kernel,
        out_type=jax.ShapeDtypeStruct((B // 8, LN, 8, 128), jnp.float32),
        mesh=mesh,
        compiler_params=pltpu.CompilerParams(
            use_tc_tiling_on_sc=False, needs_layout_passes=False
        ),
        scratch_types=[
            pltpu.VMEM((B * P // NS,), jnp.float32),
            pltpu.VMEM((B * P // NS,), jnp.int32),
            pltpu.VMEM((P * LN,), jnp.int32),
            pltpu.VMEM((B * P,), jnp.int32),
            pltpu.VMEM((P // 16, 8, DW), jnp.float32),
            pltpu.VMEM((L // 8, 8, DW), jnp.float32),
            pltpu.VMEM((L, LN), jnp.int32),
            pltpu.VMEM((B // 8, 8, DW), jnp.float32),
            pltpu.VMEM((B // 2, 8, LN), jnp.int32),
            pltpu.VMEM_SHARED((B * P,), jnp.int32),
            pltpu.SemaphoreType.DMA,
        ],
    )(_sc_body)
    return f(xf, pos4, val4)


def _classify_body(enc_ref, w_ref, out_ref):
    out_ref[...] = lax.dot_general(
        enc_ref[...],
        w_ref[...],
        (((1,), (1,)), ((), ())),
        preferred_element_type=jnp.float32,
    )


@jax.jit
def _classify(enc, classify_weight):
    return pl.pallas_call(
        _classify_body,
        out_shape=jax.ShapeDtypeStruct((B, C), jnp.float32),
    )(enc, classify_weight)


def kernel(x, position_weight, value_weight, classify_weight):
    xf = x.reshape(B * P)
    pos4 = position_weight.reshape(P // 8, 8, LN, 128).transpose(0, 2, 1, 3)
    val4 = value_weight.reshape(L // 8, 8, LN, 128).transpose(0, 2, 1, 3)
    enc4 = _sc_encode(xf, pos4, val4)
    enc = enc4.transpose(0, 2, 1, 3).reshape(B, D)
    return _classify(enc, classify_weight)


# speedup vs baseline: 1.9167x; 1.1176x over previous
import functools

import numpy as np

import jax
import jax.numpy as jnp
from jax import lax
from jax.experimental import pallas as pl
from jax.experimental.pallas import tpu as pltpu
from jax.experimental.pallas import tpu_sc as plsc

B, P, D, L, C = 32, 784, 2048, 256, 100
NC, NS, LN = 2, 16, 16
NW = NC * NS
DW = D // NW
BBLK = 16
NPAIR = BBLK // 2
L1N, L2N, L2C = 14, 14, 4

_GDN = lax.GatherDimensionNumbers(
    offset_dims=(), collapsed_slice_dims=(0,), start_index_map=(0,)
)


def _vreg_take(vec, lanes):
    return lax.gather(
        vec,
        lanes[:, None],
        _GDN,
        slice_sizes=(1,),
        mode=lax.GatherScatterMode.PROMISE_IN_BOUNDS,
    )


def _sc_body(x_hbm, pos_hbm, val_hbm, out_hbm, xs_v, qtmp_v, posw_v, idx_v,
             posraw_v, valraw_v, val_v, enc_v, wacc_v, spidx_v, sem):
    c = lax.axis_index("c")
    s = lax.axis_index("s")
    wid = s * NC + c
    cb = wid // 2
    h = (wid % 2) * DW

    PH = P // 16
    cp_pos = pltpu.async_copy(
        pos_hbm.at[pl.ds(0, PH), cb, :, pl.ds(h, DW)], posraw_v, sem
    )

    XS = B * P // NS
    pltpu.sync_copy(x_hbm.at[pl.ds(s * XS, XS)], xs_v)

    def qbody(i, _):
        v = xs_v[pl.ds(i * LN, LN)] * jnp.float32(L - 1)
        t = v + jnp.float32(0.5)
        ii = t.astype(jnp.int32)
        tie = (ii.astype(jnp.float32) == t) & ((ii & 1) == 1)
        ii = jnp.where(tie, ii - 1, ii)
        ii = jnp.clip(ii, 0, L - 1)
        qtmp_v[pl.ds(i * LN, LN)] = ii
        return 0

    lax.fori_loop(0, XS // LN, qbody, 0)
    pltpu.sync_copy(qtmp_v, spidx_v.at[pl.ds(s * XS, XS)])
    plsc.subcore_barrier()
    pltpu.sync_copy(spidx_v, idx_v)
    cp_pos.wait()
    cp_val = pltpu.async_copy(
        val_hbm.at[:, cb, :, pl.ds(h, DW)], valraw_v, sem
    )
    PHALF = P // 2

    iota = lax.iota(jnp.int32, LN)
    one = jnp.full((LN,), 1, jnp.int32)
    zero16 = jnp.zeros((LN,), jnp.int32)
    swap_pat = (iota + 8) & 15
    dup_pat = iota & 7

    def _pack_row(raw_ref, rb, r):
        w = zero16
        for n2 in range(4):
            v = raw_ref[rb, r, pl.ds(n2 * LN, LN)]
            bit = jnp.where(v < 0, one, zero16)
            nib = bit | (_vreg_take(bit, swap_pat) << 4)
            w = w | (nib << (8 * n2))
        return _vreg_take(w, dup_pat)

    def posbody(p, _):
        posw_v[pl.ds(p * LN, LN)] = _pack_row(posraw_v, p >> 3, p & 7)
        return 0

    lax.fori_loop(0, PHALF, posbody, 0)
    cp_pos2 = pltpu.async_copy(
        pos_hbm.at[pl.ds(PH, PH), cb, :, pl.ds(h, DW)], posraw_v, sem
    )
    cp_pos2.wait()

    def posbody2(p, _):
        posw_v[pl.ds(p * LN, LN)] = _pack_row(posraw_v, (p >> 3) - PH, p & 7)
        return 0

    lax.fori_loop(PHALF, P, posbody2, 0)
    cp_val.wait()

    def valbody(l, _):
        val_v[l, :] = _pack_row(valraw_v, l >> 3, l & 7)
        return 0

    lax.fori_loop(0, L, valbody, 0)

    nib_mask = jnp.full((LN,), 0x0F0F0F0F, jnp.int32)
    byte_mask = jnp.full((LN,), 0xFF, jnp.int32)
    halfsel = iota >> 3
    pair_pat = [jnp.int32(2 * j) + halfsel for j in range(NPAIR)]

    for bb in range(B // BBLK):
        b_flat = (jnp.int32(bb * BBLK) + iota) * jnp.int32(P)
        for j in range(NPAIR):
            for n in range(8):
                wacc_v[bb * NPAIR + j, n, :] = zero16

        def l2body(l2, _):
            def l1body(l1, byteaccs):
                base = l2 * (L1N * L2N) + l1 * L1N

                def pbody(i, nibaccs):
                    p = base + i
                    idxrow = plsc.load_gather(idx_v, [b_flat + p])
                    posw = posw_v[pl.ds(p * LN, LN)]
                    out = []
                    for j in range(NPAIR):
                        rbp = _vreg_take(idxrow, pair_pat[j])
                        valw = plsc.load_gather(val_v, [rbp, iota])
                        out.append(nibaccs[j] + (valw ^ posw))
                    return tuple(out)

                nib = lax.fori_loop(
                    0, L1N, pbody, tuple(zero16 for _ in range(NPAIR))
                )
                out = []
                for j in range(NPAIR):
                    lo, hi = byteaccs[2 * j], byteaccs[2 * j + 1]
                    out.append(lo + (nib[j] & nib_mask))
                    out.append(hi + ((nib[j] >> 4) & nib_mask))
                return tuple(out)

            byteaccs = lax.fori_loop(
                0, L2N, l1body, tuple(zero16 for _ in range(2 * NPAIR))
            )
            for j in range(NPAIR):
                row = bb * NPAIR + j
                lo, hi = byteaccs[2 * j], byteaccs[2 * j + 1]
                for r in range(4):
                    wacc_v[row, 2 * r, :] = (
                        wacc_v[row, 2 * r, :]
                        + (lax.shift_right_logical(lo, jnp.int32(8 * r)) & byte_mask)
                    )
                    wacc_v[row, 2 * r + 1, :] = (
                        wacc_v[row, 2 * r + 1, :]
                        + (lax.shift_right_logical(hi, jnp.int32(8 * r)) & byte_mask)
                    )
            return 0

        lax.fori_loop(0, L2C, l2body, 0)

    half = jnp.int32(P // 2)
    col8 = iota & 7
    for q in range(B // 2):
        bvec = jnp.int32(2 * q) + halfsel
        rb = bvec >> 3
        rr = bvec & 7
        for n in range(8):
            cnt = wacc_v[q, n, :]
            e = jnp.where(cnt < half, jnp.float32(1), jnp.float32(-1))
            plsc.store_scatter(enc_v, [rb, rr, col8 + jnp.int32(8 * n)], e)

    pltpu.sync_copy(enc_v, out_hbm.at[:, cb, :, pl.ds(h, DW)])


@jax.jit
def _sc_encode(xf, pos4, val4):
    mesh = plsc.VectorSubcoreMesh(core_axis_name="c", subcore_axis_name="s")
    f = functools.partial(
        pl.kernel,
        out_type=jax.ShapeDtypeStruct((B // 8, LN, 8, 128), jnp.float32),
        mesh=mesh,
        compiler_params=pltpu.CompilerParams(
            use_tc_tiling_on_sc=False, needs_layout_passes=False
        ),
        scratch_types=[
            pltpu.VMEM((B * P // NS,), jnp.float32),
            pltpu.VMEM((B * P // NS,), jnp.int32),
            pltpu.VMEM((P * LN,), jnp.int32),
            pltpu.VMEM((B * P,), jnp.int32),
            pltpu.VMEM((P // 16, 8, DW), jnp.float32),
            pltpu.VMEM((L // 8, 8, DW), jnp.float32),
            pltpu.VMEM((L, LN), jnp.int32),
            pltpu.VMEM((B // 8, 8, DW), jnp.float32),
            pltpu.VMEM((B // 2, 8, LN), jnp.int32),
            pltpu.VMEM_SHARED((B * P,), jnp.int32),
            pltpu.SemaphoreType.DMA,
        ],
    )(_sc_body)
    return f(xf, pos4, val4)


def _classify_body(enc_ref, w_ref, out_ref):
    out_ref[...] = lax.dot_general(
        enc_ref[...],
        w_ref[...],
        (((1,), (1,)), ((), ())),
        preferred_element_type=jnp.float32,
    )


@jax.jit
def _classify(enc, classify_weight):
    return pl.pallas_call(
        _classify_body,
        out_shape=jax.ShapeDtypeStruct((B, C), jnp.float32),
    )(enc, classify_weight)


def kernel(x, position_weight, value_weight, classify_weight):
    xf = x.reshape(B * P)
    pos4 = position_weight.reshape(P // 8, 8, LN, 128).transpose(0, 2, 1, 3)
    val4 = value_weight.reshape(L // 8, 8, LN, 128).transpose(0, 2, 1, 3)
    enc4 = _sc_encode(xf, pos4, val4)
    enc = enc4.transpose(0, 2, 1, 3).reshape(B, D)
    return _classify(enc, classify_weight)


# speedup vs baseline: 1.9632x; 1.0242x over previous
import functools

import numpy as np

import jax
import jax.numpy as jnp
from jax import lax
from jax.experimental import pallas as pl
from jax.experimental.pallas import tpu as pltpu
from jax.experimental.pallas import tpu_sc as plsc

B, P, D, L, C = 32, 784, 2048, 256, 100
NC, NS, LN = 2, 16, 16
NW = NC * NS
DW = D // NW
BBLK = 16
NPAIR = BBLK // 2
L1N, L2N, L2C = 14, 14, 4

_GDN = lax.GatherDimensionNumbers(
    offset_dims=(), collapsed_slice_dims=(0,), start_index_map=(0,)
)


def _vreg_take(vec, lanes):
    return lax.gather(
        vec,
        lanes[:, None],
        _GDN,
        slice_sizes=(1,),
        mode=lax.GatherScatterMode.PROMISE_IN_BOUNDS,
    )


def _sc_body(x_hbm, pos_hbm, val_hbm, out_hbm, xs_v, qtmp_v, posw_v, idx_v,
             posraw_v, valraw_v, val_v, enc_v, wacc_v, spidx_v, sem):
    c = lax.axis_index("c")
    s = lax.axis_index("s")
    wid = s * NC + c
    cb = wid // 2
    h = (wid % 2) * DW

    PH = P // 16
    cp_pos = pltpu.async_copy(
        pos_hbm.at[pl.ds(0, PH), cb, :, pl.ds(h, DW)], posraw_v, sem
    )

    XS = B * P // NS
    pltpu.sync_copy(x_hbm.at[pl.ds(s * XS, XS)], xs_v)

    def qbody(i, _):
        v = xs_v[pl.ds(i * LN, LN)] * jnp.float32(L - 1)
        t = v + jnp.float32(0.5)
        ii = t.astype(jnp.int32)
        tie = (ii.astype(jnp.float32) == t) & ((ii & 1) == 1)
        ii = jnp.where(tie, ii - 1, ii)
        ii = jnp.clip(ii, 0, L - 1)
        qtmp_v[pl.ds(i * LN, LN)] = ii << 4
        return 0

    lax.fori_loop(0, XS // LN, qbody, 0)
    pltpu.sync_copy(qtmp_v, spidx_v.at[pl.ds(s * XS, XS)])
    plsc.subcore_barrier()
    pltpu.sync_copy(spidx_v, idx_v)
    cp_pos.wait()
    cp_val = pltpu.async_copy(
        val_hbm.at[:, cb, :, pl.ds(h, DW)], valraw_v, sem
    )
    PHALF = P // 2

    iota = lax.iota(jnp.int32, LN)
    one = jnp.full((LN,), 1, jnp.int32)
    zero16 = jnp.zeros((LN,), jnp.int32)
    swap_pat = (iota + 8) & 15
    dup_pat = iota & 7

    def _pack_row(raw_ref, rb, r):
        w = zero16
        for n2 in range(4):
            v = raw_ref[rb, r, pl.ds(n2 * LN, LN)]
            bit = jnp.where(v < 0, one, zero16)
            nib = bit | (_vreg_take(bit, swap_pat) << 4)
            w = w | (nib << (8 * n2))
        return _vreg_take(w, dup_pat)

    def posbody(p, _):
        posw_v[pl.ds(p * LN, LN)] = _pack_row(posraw_v, p >> 3, p & 7)
        return 0

    lax.fori_loop(0, PHALF, posbody, 0)
    cp_pos2 = pltpu.async_copy(
        pos_hbm.at[pl.ds(PH, PH), cb, :, pl.ds(h, DW)], posraw_v, sem
    )
    cp_pos2.wait()

    def posbody2(p, _):
        posw_v[pl.ds(p * LN, LN)] = _pack_row(posraw_v, (p >> 3) - PH, p & 7)
        return 0

    lax.fori_loop(PHALF, P, posbody2, 0)
    cp_val.wait()

    def valbody(l, _):
        val_v[pl.ds(l * LN, LN)] = _pack_row(valraw_v, l >> 3, l & 7)
        return 0

    lax.fori_loop(0, L, valbody, 0)

    nib_mask = jnp.full((LN,), 0x0F0F0F0F, jnp.int32)
    byte_mask = jnp.full((LN,), 0xFF, jnp.int32)
    halfsel = iota >> 3
    pair_pat = [jnp.int32(2 * j) + halfsel for j in range(NPAIR)]

    for bb in range(B // BBLK):
        b_flat = (jnp.int32(bb * BBLK) + iota) * jnp.int32(P)
        for j in range(NPAIR):
            for n in range(8):
                wacc_v[bb * NPAIR + j, n, :] = zero16

        def l2body(l2, _):
            def l1body(l1, byteaccs):
                base = l2 * (L1N * L2N) + l1 * L1N

                def pbody(i, nibaccs):
                    p = base + i
                    idxrow = plsc.load_gather(idx_v, [b_flat + p])
                    posw = posw_v[pl.ds(p * LN, LN)]
                    out = []
                    for j in range(NPAIR):
                        rbp = _vreg_take(idxrow, pair_pat[j])
                        valw = plsc.load_gather(val_v, [rbp | iota])
                        out.append(nibaccs[j] + (valw ^ posw))
                    return tuple(out)

                nib = lax.fori_loop(
                    0, L1N, pbody, tuple(zero16 for _ in range(NPAIR))
                )
                out = []
                for j in range(NPAIR):
                    lo, hi = byteaccs[2 * j], byteaccs[2 * j + 1]
                    out.append(lo + (nib[j] & nib_mask))
                    out.append(hi + ((nib[j] >> 4) & nib_mask))
                return tuple(out)

            byteaccs = lax.fori_loop(
                0, L2N, l1body, tuple(zero16 for _ in range(2 * NPAIR))
            )
            for j in range(NPAIR):
                row = bb * NPAIR + j
                lo, hi = byteaccs[2 * j], byteaccs[2 * j + 1]
                for r in range(4):
                    wacc_v[row, 2 * r, :] = (
                        wacc_v[row, 2 * r, :]
                        + (lax.shift_right_logical(lo, jnp.int32(8 * r)) & byte_mask)
                    )
                    wacc_v[row, 2 * r + 1, :] = (
                        wacc_v[row, 2 * r + 1, :]
                        + (lax.shift_right_logical(hi, jnp.int32(8 * r)) & byte_mask)
                    )
            return 0

        lax.fori_loop(0, L2C, l2body, 0)

    half = jnp.int32(P // 2)
    col8 = iota & 7
    for q in range(B // 2):
        bvec = jnp.int32(2 * q) + halfsel
        rb = bvec >> 3
        rr = bvec & 7
        for n in range(8):
            cnt = wacc_v[q, n, :]
            e = jnp.where(cnt < half, jnp.float32(1), jnp.float32(-1))
            plsc.store_scatter(enc_v, [rb, rr, col8 + jnp.int32(8 * n)], e)

    pltpu.sync_copy(enc_v, out_hbm.at[:, cb, :, pl.ds(h, DW)])


@jax.jit
def _sc_encode(xf, pos4, val4):
    mesh = plsc.VectorSubcoreMesh(core_axis_name="c", subcore_axis_name="s")
    f = functools.partial(
        pl.kernel,
        out_type=jax.ShapeDtypeStruct((B // 8, LN, 8, 128), jnp.float32),
        mesh=mesh,
        compiler_params=pltpu.CompilerParams(
            use_tc_tiling_on_sc=False, needs_layout_passes=False
        ),
        scratch_types=[
            pltpu.VMEM((B * P // NS,), jnp.float32),
            pltpu.VMEM((B * P // NS,), jnp.int32),
            pltpu.VMEM((P * LN,), jnp.int32),
            pltpu.VMEM((B * P,), jnp.int32),
            pltpu.VMEM((P // 16, 8, DW), jnp.float32),
            pltpu.VMEM((L // 8, 8, DW), jnp.float32),
            pltpu.VMEM((L * LN,), jnp.int32),
            pltpu.VMEM((B // 8, 8, DW), jnp.float32),
            pltpu.VMEM((B // 2, 8, LN), jnp.int32),
            pltpu.VMEM_SHARED((B * P,), jnp.int32),
            pltpu.SemaphoreType.DMA,
        ],
    )(_sc_body)
    return f(xf, pos4, val4)


def _classify_body(enc_ref, w_ref, out_ref):
    out_ref[...] = lax.dot_general(
        enc_ref[...],
        w_ref[...],
        (((1,), (1,)), ((), ())),
        preferred_element_type=jnp.float32,
    )


@jax.jit
def _classify(enc, classify_weight):
    return pl.pallas_call(
        _classify_body,
        out_shape=jax.ShapeDtypeStruct((B, C), jnp.float32),
    )(enc, classify_weight)


def kernel(x, position_weight, value_weight, classify_weight):
    xf = x.reshape(B * P)
    pos4 = position_weight.reshape(P // 8, 8, LN, 128).transpose(0, 2, 1, 3)
    val4 = value_weight.reshape(L // 8, 8, LN, 128).transpose(0, 2, 1, 3)
    enc4 = _sc_encode(xf, pos4, val4)
    enc = enc4.transpose(0, 2, 1, 3).reshape(B, D)
    return _classify(enc, classify_weight)


# speedup vs baseline: 2.0774x; 1.0582x over previous
import functools

import numpy as np

import jax
import jax.numpy as jnp
from jax import lax
from jax.experimental import pallas as pl
from jax.experimental.pallas import tpu as pltpu
from jax.experimental.pallas import tpu_sc as plsc

B, P, D, L, C = 32, 784, 2048, 256, 100
NC, NS, LN = 2, 16, 16
NW = NC * NS
DW = D // NW
BBLK = 16
NPAIR = BBLK // 2
L1N, L2N, L2C = 14, 14, 4

_GDN = lax.GatherDimensionNumbers(
    offset_dims=(), collapsed_slice_dims=(0,), start_index_map=(0,)
)


def _vreg_take(vec, lanes):
    return lax.gather(
        vec,
        lanes[:, None],
        _GDN,
        slice_sizes=(1,),
        mode=lax.GatherScatterMode.PROMISE_IN_BOUNDS,
    )


def _sc_body(x_hbm, pos_hbm, val_hbm, out_hbm, xs_v, qtmp_v, posw_v, idx_v,
             posraw_v, valraw_v, val_v, enc_v, wacc_v, spidx_v, sem):
    c = lax.axis_index("c")
    s = lax.axis_index("s")
    wid = s * NC + c
    cb = wid // 2
    h = (wid % 2) * DW

    PH = P // 16
    cp_pos = pltpu.async_copy(
        pos_hbm.at[pl.ds(0, PH), cb, :, pl.ds(h, DW)], posraw_v, sem
    )

    XS = B * P // NS
    pltpu.sync_copy(x_hbm.at[pl.ds(s * XS, XS)], xs_v)

    def qbody(i, _):
        v = xs_v[pl.ds(i * LN, LN)] * jnp.float32(L - 1)
        t = v + jnp.float32(0.5)
        ii = t.astype(jnp.int32)
        tie = (ii.astype(jnp.float32) == t) & ((ii & 1) == 1)
        ii = jnp.where(tie, ii - 1, ii)
        ii = jnp.clip(ii, 0, L - 1)
        qtmp_v[pl.ds(i * LN, LN)] = ii << 4
        return 0

    lax.fori_loop(0, XS // LN, qbody, 0)
    pltpu.sync_copy(qtmp_v, spidx_v.at[pl.ds(s * XS, XS)])
    plsc.subcore_barrier()
    pltpu.sync_copy(spidx_v, idx_v)
    cp_pos.wait()
    cp_val = pltpu.async_copy(
        val_hbm.at[:, cb, :, pl.ds(h, DW)], valraw_v, sem
    )
    PHALF = P // 2

    iota = lax.iota(jnp.int32, LN)
    one = jnp.full((LN,), 1, jnp.int32)
    zero16 = jnp.zeros((LN,), jnp.int32)
    swap_pat = (iota + 8) & 15
    dup_pat = iota & 7

    halfsel0 = iota >> 3
    shbits = [one << (jnp.int32(8 * n2) + 4 * halfsel0) for n2 in range(4)]

    def _pack_row(raw_ref, rb, r):
        w = zero16
        for n2 in range(4):
            v = raw_ref[rb, r, pl.ds(n2 * LN, LN)]
            w = w | jnp.where(v < 0, shbits[n2], zero16)
        return w | _vreg_take(w, swap_pat)

    def posbody(p, _):
        posw_v[pl.ds(p * LN, LN)] = _pack_row(posraw_v, p >> 3, p & 7)
        return 0

    lax.fori_loop(0, PHALF, posbody, 0)
    cp_pos2 = pltpu.async_copy(
        pos_hbm.at[pl.ds(PH, PH), cb, :, pl.ds(h, DW)], posraw_v, sem
    )
    cp_pos2.wait()

    def posbody2(p, _):
        posw_v[pl.ds(p * LN, LN)] = _pack_row(posraw_v, (p >> 3) - PH, p & 7)
        return 0

    lax.fori_loop(PHALF, P, posbody2, 0)
    cp_val.wait()

    def valbody(l, _):
        val_v[pl.ds(l * LN, LN)] = _pack_row(valraw_v, l >> 3, l & 7)
        return 0

    lax.fori_loop(0, L, valbody, 0)

    nib_mask = jnp.full((LN,), 0x0F0F0F0F, jnp.int32)
    byte_mask = jnp.full((LN,), 0xFF, jnp.int32)
    halfsel = iota >> 3
    pair_pat = [jnp.int32(2 * j) + halfsel for j in range(NPAIR)]

    for bb in range(B // BBLK):
        b_flat = (jnp.int32(bb * BBLK) + iota) * jnp.int32(P)
        for j in range(NPAIR):
            for n in range(8):
                wacc_v[bb * NPAIR + j, n, :] = zero16

        def l2body(l2, _):
            def l1body(l1, byteaccs):
                base = l2 * (L1N * L2N) + l1 * L1N

                def pbody(i, nibaccs):
                    p = base + i
                    idxrow = plsc.load_gather(idx_v, [b_flat + p])
                    posw = posw_v[pl.ds(p * LN, LN)]
                    out = []
                    for j in range(NPAIR):
                        rbp = _vreg_take(idxrow, pair_pat[j])
                        valw = plsc.load_gather(val_v, [rbp | iota])
                        out.append(nibaccs[j] + (valw ^ posw))
                    return tuple(out)

                nib = lax.fori_loop(
                    0, L1N, pbody, tuple(zero16 for _ in range(NPAIR))
                )
                out = []
                for j in range(NPAIR):
                    lo, hi = byteaccs[2 * j], byteaccs[2 * j + 1]
                    out.append(lo + (nib[j] & nib_mask))
                    out.append(hi + ((nib[j] >> 4) & nib_mask))
                return tuple(out)

            byteaccs = lax.fori_loop(
                0, L2N, l1body, tuple(zero16 for _ in range(2 * NPAIR))
            )
            for j in range(NPAIR):
                row = bb * NPAIR + j
                lo, hi = byteaccs[2 * j], byteaccs[2 * j + 1]
                for r in range(4):
                    wacc_v[row, 2 * r, :] = (
                        wacc_v[row, 2 * r, :]
                        + (lax.shift_right_logical(lo, jnp.int32(8 * r)) & byte_mask)
                    )
                    wacc_v[row, 2 * r + 1, :] = (
                        wacc_v[row, 2 * r + 1, :]
                        + (lax.shift_right_logical(hi, jnp.int32(8 * r)) & byte_mask)
                    )
            return 0

        lax.fori_loop(0, L2C, l2body, 0)

    half = jnp.int32(P // 2)
    col8 = iota & 7
    for q in range(B // 2):
        bvec = jnp.int32(2 * q) + halfsel
        rb = bvec >> 3
        rr = bvec & 7
        for n in range(8):
            cnt = wacc_v[q, n, :]
            e = jnp.where(cnt < half, jnp.float32(1), jnp.float32(-1))
            plsc.store_scatter(enc_v, [rb, rr, col8 + jnp.int32(8 * n)], e)

    pltpu.sync_copy(enc_v, out_hbm.at[:, cb, :, pl.ds(h, DW)])


@jax.jit
def _sc_encode(xf, pos4, val4):
    mesh = plsc.VectorSubcoreMesh(core_axis_name="c", subcore_axis_name="s")
    f = functools.partial(
        pl.kernel,
        out_type=jax.ShapeDtypeStruct((B // 8, LN, 8, 128), jnp.float32),
        mesh=mesh,
        compiler_params=pltpu.CompilerParams(
            use_tc_tiling_on_sc=False, needs_layout_passes=False
        ),
        scratch_types=[
            pltpu.VMEM((B * P // NS,), jnp.float32),
            pltpu.VMEM((B * P // NS,), jnp.int32),
            pltpu.VMEM((P * LN,), jnp.int32),
            pltpu.VMEM((B * P,), jnp.int32),
            pltpu.VMEM((P // 16, 8, DW), jnp.float32),
            pltpu.VMEM((L // 8, 8, DW), jnp.float32),
            pltpu.VMEM((L * LN,), jnp.int32),
            pltpu.VMEM((B // 8, 8, DW), jnp.float32),
            pltpu.VMEM((B // 2, 8, LN), jnp.int32),
            pltpu.VMEM_SHARED((B * P,), jnp.int32),
            pltpu.SemaphoreType.DMA,
        ],
    )(_sc_body)
    return f(xf, pos4, val4)


def _classify_body(enc_ref, w_ref, out_ref):
    out_ref[...] = lax.dot_general(
        enc_ref[...],
        w_ref[...],
        (((1,), (1,)), ((), ())),
        preferred_element_type=jnp.float32,
    )


@jax.jit
def _classify(enc, classify_weight):
    return pl.pallas_call(
        _classify_body,
        out_shape=jax.ShapeDtypeStruct((B, C), jnp.float32),
    )(enc, classify_weight)


def kernel(x, position_weight, value_weight, classify_weight):
    xf = x.reshape(B * P)
    pos4 = position_weight.reshape(P // 8, 8, LN, 128).transpose(0, 2, 1, 3)
    val4 = value_weight.reshape(L // 8, 8, LN, 128).transpose(0, 2, 1, 3)
    enc4 = _sc_encode(xf, pos4, val4)
    enc = enc4.transpose(0, 2, 1, 3).reshape(B, D)
    return _classify(enc, classify_weight)


# speedup vs baseline: 2.0783x; 1.0005x over previous
import functools

import jax
import jax.numpy as jnp
from jax import lax
from jax.experimental import pallas as pl
from jax.experimental.pallas import tpu as pltpu
from jax.experimental.pallas import tpu_sc as plsc

B, P, D, L, C = 32, 784, 2048, 256, 100
NC, NS, LN = 2, 16, 16
NW = NC * NS
DW = D // NW
BBLK = 16
NPAIR = BBLK // 2
L1N, L2N, L2C = 14, 14, 4

_GDN = lax.GatherDimensionNumbers(
    offset_dims=(), collapsed_slice_dims=(0,), start_index_map=(0,)
)


def _vreg_take(vec, lanes):
    return lax.gather(
        vec,
        lanes[:, None],
        _GDN,
        slice_sizes=(1,),
        mode=lax.GatherScatterMode.PROMISE_IN_BOUNDS,
    )


def _sc_body(x_hbm, pos_hbm, val_hbm, out_hbm, xs_v, qtmp_v, posw_v, idx_v,
             posraw_v, valraw_v, val_v, enc_v, wacc_v, spidx_v, sem):
    c = lax.axis_index("c")
    s = lax.axis_index("s")
    wid = s * NC + c
    cb = wid // 2
    h = (wid % 2) * DW

    PH = P // 16
    cp_pos = pltpu.async_copy(
        pos_hbm.at[pl.ds(0, PH), cb, :, pl.ds(h, DW)], posraw_v, sem
    )

    XS = B * P // NS
    pltpu.sync_copy(x_hbm.at[pl.ds(s * XS, XS)], xs_v)

    def qbody(i, _):
        v = xs_v[pl.ds(i * LN, LN)] * jnp.float32(L - 1)
        t = v + jnp.float32(0.5)
        ii = t.astype(jnp.int32)
        tie = (ii.astype(jnp.float32) == t) & ((ii & 1) == 1)
        ii = jnp.where(tie, ii - 1, ii)
        ii = jnp.clip(ii, 0, L - 1)
        qtmp_v[pl.ds(i * LN, LN)] = ii << 4
        return 0

    lax.fori_loop(0, XS // LN, qbody, 0)
    pltpu.sync_copy(qtmp_v, spidx_v.at[pl.ds(s * XS, XS)])
    plsc.subcore_barrier()
    pltpu.sync_copy(spidx_v, idx_v)
    cp_pos.wait()
    cp_val = pltpu.async_copy(
        val_hbm.at[:, cb, :, pl.ds(h, DW)], valraw_v, sem
    )
    PHALF = P // 2

    iota = lax.iota(jnp.int32, LN)
    one = jnp.full((LN,), 1, jnp.int32)
    zero16 = jnp.zeros((LN,), jnp.int32)
    swap_pat = (iota + 8) & 15

    halfsel0 = iota >> 3
    shbits = [one << (jnp.int32(8 * n2) + 4 * halfsel0) for n2 in range(4)]

    def _pack_row(raw_ref, rb, r):
        w = zero16
        for n2 in range(4):
            v = raw_ref[rb, r, pl.ds(n2 * LN, LN)]
            w = w | jnp.where(v < 0, shbits[n2], zero16)
        return w | _vreg_take(w, swap_pat)

    def posbody(p, _):
        posw_v[pl.ds(p * LN, LN)] = _pack_row(posraw_v, p >> 3, p & 7)
        return 0

    lax.fori_loop(0, PHALF, posbody, 0)
    cp_pos2 = pltpu.async_copy(
        pos_hbm.at[pl.ds(PH, PH), cb, :, pl.ds(h, DW)], posraw_v, sem
    )
    cp_pos2.wait()

    def posbody2(p, _):
        posw_v[pl.ds(p * LN, LN)] = _pack_row(posraw_v, (p >> 3) - PH, p & 7)
        return 0

    lax.fori_loop(PHALF, P, posbody2, 0)
    cp_val.wait()

    def valbody(l, _):
        val_v[pl.ds(l * LN, LN)] = _pack_row(valraw_v, l >> 3, l & 7)
        return 0

    lax.fori_loop(0, L, valbody, 0)

    nib_mask = jnp.full((LN,), 0x0F0F0F0F, jnp.int32)
    byte_mask = jnp.full((LN,), 0xFF, jnp.int32)
    halfsel = iota >> 3
    pair_pat = [jnp.int32(2 * j) + halfsel for j in range(NPAIR)]

    for bb in range(B // BBLK):
        b_flat = (jnp.int32(bb * BBLK) + iota) * jnp.int32(P)
        for j in range(NPAIR):
            for n in range(8):
                wacc_v[bb * NPAIR + j, n, :] = zero16

        def l2body(l2, _):
            def l1body(l1, byteaccs):
                base = l2 * (L1N * L2N) + l1 * L1N

                def pbody(i, nibaccs):
                    p = base + i
                    idxrow = plsc.load_gather(idx_v, [b_flat + p])
                    posw = posw_v[pl.ds(p * LN, LN)]
                    out = []
                    for j in range(NPAIR):
                        rbp = _vreg_take(idxrow, pair_pat[j])
                        valw = plsc.load_gather(val_v, [rbp | iota])
                        out.append(nibaccs[j] + (valw ^ posw))
                    return tuple(out)

                nib = lax.fori_loop(
                    0, L1N, pbody, tuple(zero16 for _ in range(NPAIR))
                )
                out = []
                for j in range(NPAIR):
                    lo, hi = byteaccs[2 * j], byteaccs[2 * j + 1]
                    out.append(lo + (nib[j] & nib_mask))
                    out.append(hi + ((nib[j] >> 4) & nib_mask))
                return tuple(out)

            byteaccs = lax.fori_loop(
                0, L2N, l1body, tuple(zero16 for _ in range(2 * NPAIR))
            )
            for j in range(NPAIR):
                row = bb * NPAIR + j
                lo, hi = byteaccs[2 * j], byteaccs[2 * j + 1]
                for r in range(4):
                    wacc_v[row, 2 * r, :] = (
                        wacc_v[row, 2 * r, :]
                        + (lax.shift_right_logical(lo, jnp.int32(8 * r)) & byte_mask)
                    )
                    wacc_v[row, 2 * r + 1, :] = (
                        wacc_v[row, 2 * r + 1, :]
                        + (lax.shift_right_logical(hi, jnp.int32(8 * r)) & byte_mask)
                    )
            return 0

        lax.fori_loop(0, L2C, l2body, 0)

    half = jnp.int32(P // 2)
    col8 = iota & 7
    for q in range(B // 2):
        bvec = jnp.int32(2 * q) + halfsel
        rb = bvec >> 3
        rr = bvec & 7
        for n in range(8):
            cnt = wacc_v[q, n, :]
            e = jnp.where(cnt < half, jnp.float32(1), jnp.float32(-1))
            plsc.store_scatter(enc_v, [rb, rr, col8 + jnp.int32(8 * n)], e)

    pltpu.sync_copy(enc_v, out_hbm.at[:, cb, :, pl.ds(h, DW)])


@jax.jit
def _sc_encode(xf, pos4, val4):
    mesh = plsc.VectorSubcoreMesh(core_axis_name="c", subcore_axis_name="s")
    f = functools.partial(
        pl.kernel,
        out_type=jax.ShapeDtypeStruct((B // 8, LN, 8, 128), jnp.float32),
        mesh=mesh,
        compiler_params=pltpu.CompilerParams(
            use_tc_tiling_on_sc=False, needs_layout_passes=False
        ),
        scratch_types=[
            pltpu.VMEM((B * P // NS,), jnp.float32),
            pltpu.VMEM((B * P // NS,), jnp.int32),
            pltpu.VMEM((P * LN,), jnp.int32),
            pltpu.VMEM((B * P,), jnp.int32),
            pltpu.VMEM((P // 16, 8, DW), jnp.float32),
            pltpu.VMEM((L // 8, 8, DW), jnp.float32),
            pltpu.VMEM((L * LN,), jnp.int32),
            pltpu.VMEM((B // 8, 8, DW), jnp.float32),
            pltpu.VMEM((B // 2, 8, LN), jnp.int32),
            pltpu.VMEM_SHARED((B * P,), jnp.int32),
            pltpu.SemaphoreType.DMA,
        ],
    )(_sc_body)
    return f(xf, pos4, val4)


def _classify_body(enc_ref, w_ref, out_ref):
    out_ref[...] = lax.dot_general(
        enc_ref[...],
        w_ref[...],
        (((1,), (1,)), ((), ())),
        preferred_element_type=jnp.float32,
    )


@jax.jit
def _classify(enc, classify_weight):
    return pl.pallas_call(
        _classify_body,
        out_shape=jax.ShapeDtypeStruct((B, C), jnp.float32),
    )(enc, classify_weight)


def kernel(x, position_weight, value_weight, classify_weight):
    xf = x.reshape(B * P)
    pos4 = position_weight.reshape(P // 8, 8, LN, 128).transpose(0, 2, 1, 3)
    val4 = value_weight.reshape(L // 8, 8, LN, 128).transpose(0, 2, 1, 3)
    enc4 = _sc_encode(xf, pos4, val4)
    enc = enc4.transpose(0, 2, 1, 3).reshape(B, D)
    return _classify(enc, classify_weight)
